# Initial kernel scaffold; baseline (speedup 1.0000x reference)
import jax
import jax.numpy as jnp
from jax.experimental import pallas as pl


def kernel(x, edge_index, W1, att_src1, att_dst1, b1, W2, att_src2, att_dst2, b2):
    raise NotImplementedError("write your pallas kernel here")


# speedup vs baseline: 34.6377x; 34.6377x over previous
import functools

import jax
import jax.numpy as jnp
from jax import lax
from jax.experimental import pallas as pl
from jax.experimental.pallas import tpu as pltpu
from jax.experimental.pallas import tpu_sc as plsc

N = 10000
NP = 10240
E = 320000
H = 8
C = 16
D = H * C
DP = 68
KC = 80
TILES = 16
ROWS_PT = NP // TILES
E_PT = E // TILES
CHUNKS = E_PT // KC
BN = 2000
GRID = N // BN

_HI = jax.lax.Precision.HIGHEST


def _mm_body(x_ref, w_ref, asd_ref, xlA_ref, xlB_ref, adA_ref, adB_ref):
    x = x_ref[...]
    xl = jnp.dot(x, w_ref[...], precision=_HI, preferred_element_type=jnp.float32)
    sd = jnp.dot(xl, asd_ref[...], precision=_HI, preferred_element_type=jnp.float32)
    xlA_ref[...] = jnp.concatenate([xl[:, 0:64], sd[:, 0:4]], axis=1)
    xlB_ref[...] = jnp.concatenate([xl[:, 64:128], sd[:, 4:8]], axis=1)
    adA_ref[...] = sd[:, 8:12]
    adB_ref[...] = sd[:, 12:16]


def _mm_call(x, W, ASD):
    return pl.pallas_call(
        _mm_body,
        grid=(GRID,),
        in_specs=[
            pl.BlockSpec((BN, D), lambda i: (i, 0)),
            pl.BlockSpec((D, D), lambda i: (0, 0)),
            pl.BlockSpec((D, 16), lambda i: (0, 0)),
        ],
        out_specs=[
            pl.BlockSpec((BN, DP), lambda i: (i, 0)),
            pl.BlockSpec((BN, DP), lambda i: (i, 0)),
            pl.BlockSpec((BN, 4), lambda i: (i, 0)),
            pl.BlockSpec((BN, 4), lambda i: (i, 0)),
        ],
        out_shape=[
            jax.ShapeDtypeStruct((N, DP), jnp.float32),
            jax.ShapeDtypeStruct((N, DP), jnp.float32),
            jax.ShapeDtypeStruct((N, 4), jnp.float32),
            jax.ShapeDtypeStruct((N, 4), jnp.float32),
        ],
    )(x, W, ASD)


def _sc_body(src_hbm, dst_hbm, xlA_hbm, xlB_hbm, adA_hbm, adB_hbm, zin_hbm,
             out_hbm, acc, srcj, dstj, msg, adtab, gsem):
    cid = lax.axis_index("c")
    sid = lax.axis_index("s")
    r0 = sid * ROWS_PT
    pltpu.sync_copy(zin_hbm.at[pl.ds(r0, ROWS_PT)], acc.at[pl.ds(r0, ROWS_PT)])

    iota = lax.iota(jnp.int32, 16)
    i_div4 = iota // 4
    i_mod4 = iota & 3
    wcols = 64 + i_mod4
    four16 = jnp.full((16,), 4, jnp.int32)
    one16 = jnp.full((16,), 1, jnp.int32)

    def run(xl_hbm, ad_hbm):
        pltpu.sync_copy(ad_hbm, adtab)
        plsc.subcore_barrier()
        e_base = sid * E_PT

        def chunk(t, carry):
            eb = e_base + t * KC
            pltpu.sync_copy(src_hbm.at[pl.ds(eb, KC)], srcj)
            pltpu.sync_copy(dst_hbm.at[pl.ds(eb, KC)], dstj)
            pltpu.async_copy(xl_hbm.at[srcj], msg, gsem).wait()

            def wgrp(g, carry2):
                i0 = i_div4 + g * four16
                nodes = plsc.load_gather(dstj, [i0])
                a_s = plsc.load_gather(msg, [i0, wcols])
                a_d = plsc.load_gather(adtab, [nodes, i_mod4])
                al = a_s + a_d
                al = jnp.where(al >= 0.0, al, 0.2 * al)
                plsc.store_scatter(msg, [i0, wcols], jnp.exp(al))
                return carry2

            lax.fori_loop(0, KC // 4, wgrp, 0)

            def scale(k, carry2):
                k16 = one16 * k
                for h in range(4):
                    wv = plsc.load_gather(
                        msg, [k16, jnp.full((16,), 64 + h, jnp.int32)])
                    ch = iota + 16 * h
                    feat = plsc.load_gather(msg, [k16, ch])
                    plsc.store_scatter(msg, [k16, ch], feat * wv)
                return carry2

            lax.fori_loop(0, KC, scale, 0)
            pltpu.sync_copy(msg, acc.at[dstj], add=True)
            return carry

        lax.fori_loop(0, CHUNKS, chunk, 0)

    @pl.when(cid == 0)
    def _():
        run(xlA_hbm, adA_hbm)

    @pl.when(cid == 1)
    def _():
        run(xlB_hbm, adB_hbm)

    plsc.subcore_barrier()
    pltpu.sync_copy(acc.at[pl.ds(r0, ROWS_PT)],
                    out_hbm.at[cid].at[pl.ds(r0, ROWS_PT)])


def _sc_call(src1d, dst1d, xlA, xlB, adA, adB, zin):
    mesh = plsc.VectorSubcoreMesh(core_axis_name="c", subcore_axis_name="s")
    f = functools.partial(
        pl.kernel,
        out_type=jax.ShapeDtypeStruct((2, NP, DP), jnp.float32),
        mesh=mesh,
        compiler_params=pltpu.CompilerParams(
            needs_layout_passes=False, use_tc_tiling_on_sc=False),
        scratch_types=[
            pltpu.VMEM_SHARED((NP, DP), jnp.float32),
            pltpu.VMEM((KC,), jnp.int32),
            pltpu.VMEM((KC,), jnp.int32),
            pltpu.VMEM((KC, DP), jnp.float32),
            pltpu.VMEM((N, 4), jnp.float32),
            pltpu.SemaphoreType.DMA,
        ],
    )(_sc_body)
    return f(src1d, dst1d, xlA, xlB, adA, adB, zin)


def _halves(acc_ref, xlA_ref, xlB_ref, adA_ref, adB_ref, e4):
    outs = []
    for half, (xl_ref, ad_ref) in enumerate(((xlA_ref, adA_ref),
                                             (xlB_ref, adB_ref))):
        acc = acc_ref[half]
        xl = xl_ref[...]
        al = xl[:, 64:68] + ad_ref[...]
        ws = jnp.exp(jnp.where(al >= 0.0, al, 0.2 * al))
        den = acc[:, 64:68] + ws + 1e-16
        wsx = jnp.dot(ws, e4, precision=_HI, preferred_element_type=jnp.float32)
        denx = jnp.dot(den, e4, precision=_HI, preferred_element_type=jnp.float32)
        outs.append((acc[:, 0:64] + wsx * xl[:, 0:64]) / denx)
    return outs


def _f1_body(acc_ref, xlA_ref, xlB_ref, adA_ref, adB_ref, e4_ref, b_ref,
             out_ref):
    outA, outB = _halves(acc_ref, xlA_ref, xlB_ref, adA_ref, adB_ref,
                         e4_ref[...])
    h = jnp.concatenate([outA, outB], axis=1) + b_ref[...]
    out_ref[...] = jnp.where(h > 0.0, h, jnp.exp(jnp.minimum(h, 0.0)) - 1.0)


def _f2_body(acc_ref, xlA_ref, xlB_ref, adA_ref, adB_ref, e4_ref, m4_ref,
             b_ref, out_ref):
    outA, outB = _halves(acc_ref, xlA_ref, xlB_ref, adA_ref, adB_ref,
                         e4_ref[...])
    m4 = m4_ref[...]
    hs = (jnp.dot(outA, m4, precision=_HI, preferred_element_type=jnp.float32)
          + jnp.dot(outB, m4, precision=_HI, preferred_element_type=jnp.float32))
    t = hs * 0.125 + b_ref[...]
    mx = jnp.max(t, axis=1, keepdims=True)
    lse = jnp.log(jnp.sum(jnp.exp(t - mx), axis=1, keepdims=True))
    out_ref[...] = t - mx - lse


def _fin_specs(extra_in, out_w):
    in_specs = [
        pl.BlockSpec((2, BN, DP), lambda i: (0, i, 0)),
        pl.BlockSpec((BN, DP), lambda i: (i, 0)),
        pl.BlockSpec((BN, DP), lambda i: (i, 0)),
        pl.BlockSpec((BN, 4), lambda i: (i, 0)),
        pl.BlockSpec((BN, 4), lambda i: (i, 0)),
        pl.BlockSpec((4, 64), lambda i: (0, 0)),
    ] + extra_in
    return dict(
        grid=(GRID,),
        in_specs=in_specs,
        out_specs=pl.BlockSpec((BN, out_w), lambda i: (i, 0)),
        out_shape=jax.ShapeDtypeStruct((N, out_w), jnp.float32),
    )


def kernel(x, edge_index, W1, att_src1, att_dst1, b1, W2, att_src2, att_dst2,
           b2):
    src1d = edge_index[0].astype(jnp.int32)
    dst1d = edge_index[1].astype(jnp.int32)
    zin = jnp.zeros((NP, DP), jnp.float32)

    onehot = (jnp.arange(H)[None, :] == (jnp.arange(D)[:, None] // C)).astype(
        jnp.float32)
    e4 = (jnp.arange(4)[:, None] == (jnp.arange(64)[None, :] // 16)).astype(
        jnp.float32)
    m4 = (jnp.arange(16)[None, :] == (jnp.arange(64)[:, None] % 16)).astype(
        jnp.float32)

    def asd_of(att_s, att_d):
        a_s = att_s.reshape(D)[:, None] * onehot
        a_d = att_d.reshape(D)[:, None] * onehot
        return jnp.concatenate([a_s, a_d], axis=1)

    xlA, xlB, adA, adB = _mm_call(x, W1, asd_of(att_src1, att_dst1))
    acc = _sc_call(src1d, dst1d, xlA, xlB, adA, adB, zin)[:, :N, :]
    h1 = pl.pallas_call(
        _f1_body,
        **_fin_specs([pl.BlockSpec((1, D), lambda i: (0, 0))], D),
    )(acc, xlA, xlB, adA, adB, e4, b1.reshape(1, D))

    xlA2, xlB2, adA2, adB2 = _mm_call(h1, W2, asd_of(att_src2, att_dst2))
    acc2 = _sc_call(src1d, dst1d, xlA2, xlB2, adA2, adB2, zin)[:, :N, :]
    out = pl.pallas_call(
        _f2_body,
        **_fin_specs([pl.BlockSpec((64, 16), lambda i: (0, 0)),
                      pl.BlockSpec((1, 16), lambda i: (0, 0))], 16),
    )(acc2, xlA2, xlB2, adA2, adB2, e4, m4, b2.reshape(1, 16))
    return out


# speedup vs baseline: 65.2975x; 1.8852x over previous
import functools

import jax
import jax.numpy as jnp
from jax import lax
from jax.experimental import pallas as pl
from jax.experimental.pallas import tpu as pltpu
from jax.experimental.pallas import tpu_sc as plsc

N = 10000
NP = 10240
E = 320000
H = 8
C = 16
D = H * C
DP = 68
KC = 80
TILES = 16
ROWS_PT = NP // TILES
E_PT = E // TILES
CHUNKS = E_PT // KC
BN = 2000
GRID = N // BN

_HI = jax.lax.Precision.HIGHEST


def _mm_body(x_ref, w_ref, asd_ref, xlA_ref, xlB_ref, adA_ref, adB_ref):
    x = x_ref[...]
    xl = jnp.dot(x, w_ref[...], precision=_HI, preferred_element_type=jnp.float32)
    sd = jnp.dot(xl, asd_ref[...], precision=_HI, preferred_element_type=jnp.float32)
    xlA_ref[...] = jnp.concatenate([xl[:, 0:64], sd[:, 0:4]], axis=1)
    xlB_ref[...] = jnp.concatenate([xl[:, 64:128], sd[:, 4:8]], axis=1)
    adA_ref[...] = sd[:, 8:12]
    adB_ref[...] = sd[:, 12:16]


def _mm_call(x, W, ASD):
    return pl.pallas_call(
        _mm_body,
        grid=(GRID,),
        in_specs=[
            pl.BlockSpec((BN, D), lambda i: (i, 0)),
            pl.BlockSpec((D, D), lambda i: (0, 0)),
            pl.BlockSpec((D, 16), lambda i: (0, 0)),
        ],
        out_specs=[
            pl.BlockSpec((BN, DP), lambda i: (i, 0)),
            pl.BlockSpec((BN, DP), lambda i: (i, 0)),
            pl.BlockSpec((BN, 4), lambda i: (i, 0)),
            pl.BlockSpec((BN, 4), lambda i: (i, 0)),
        ],
        out_shape=[
            jax.ShapeDtypeStruct((N, DP), jnp.float32),
            jax.ShapeDtypeStruct((N, DP), jnp.float32),
            jax.ShapeDtypeStruct((N, 4), jnp.float32),
            jax.ShapeDtypeStruct((N, 4), jnp.float32),
        ],
    )(x, W, ASD)


def _sc_body(src_hbm, dst_hbm, xlA_hbm, xlB_hbm, adA_hbm, adB_hbm, zin_hbm,
             out_hbm, acc, srcj, dstj, msg, adtab, gsem):
    cid = lax.axis_index("c")
    sid = lax.axis_index("s")
    r0 = sid * ROWS_PT
    pltpu.sync_copy(zin_hbm.at[pl.ds(r0, ROWS_PT)], acc.at[pl.ds(r0, ROWS_PT)])

    iota = lax.iota(jnp.int32, 16)
    i_div4 = iota // 4
    i_mod4 = iota & 3
    wcols = 64 + i_mod4
    four16 = jnp.full((16,), 4, jnp.int32)
    one16 = jnp.full((16,), 1, jnp.int32)

    def run(xl_hbm, ad_hbm):
        pltpu.sync_copy(ad_hbm, adtab)
        plsc.subcore_barrier()
        e_base = sid * E_PT

        def chunk(t, carry):
            eb = e_base + t * KC
            pltpu.sync_copy(src_hbm.at[pl.ds(eb, KC)], srcj)
            pltpu.sync_copy(dst_hbm.at[pl.ds(eb, KC)], dstj)
            pltpu.async_copy(xl_hbm.at[srcj], msg, gsem).wait()

            @functools.partial(plsc.parallel_loop, 0, KC // 4, unroll=4)
            def _wgrp(g):
                i0 = i_div4 + g * four16
                nodes = plsc.load_gather(dstj, [i0])
                a_s = plsc.load_gather(msg, [i0, wcols])
                a_d = plsc.load_gather(adtab, [nodes, i_mod4])
                al = a_s + a_d
                al = jnp.where(al >= 0.0, al, 0.2 * al)
                plsc.store_scatter(msg, [i0, wcols], jnp.exp(al))

            @functools.partial(plsc.parallel_loop, 0, KC, unroll=4)
            def _scale(k):
                k16 = one16 * k
                for h in range(4):
                    wv = plsc.load_gather(
                        msg, [k16, jnp.full((16,), 64 + h, jnp.int32)])
                    ch = iota + 16 * h
                    feat = plsc.load_gather(msg, [k16, ch])
                    plsc.store_scatter(msg, [k16, ch], feat * wv)
            pltpu.sync_copy(msg, acc.at[dstj], add=True)
            return carry

        lax.fori_loop(0, CHUNKS, chunk, 0)

    @pl.when(cid == 0)
    def _():
        run(xlA_hbm, adA_hbm)

    @pl.when(cid == 1)
    def _():
        run(xlB_hbm, adB_hbm)

    plsc.subcore_barrier()
    pltpu.sync_copy(acc.at[pl.ds(r0, ROWS_PT)],
                    out_hbm.at[cid].at[pl.ds(r0, ROWS_PT)])


def _sc_call(src1d, dst1d, xlA, xlB, adA, adB, zin):
    mesh = plsc.VectorSubcoreMesh(core_axis_name="c", subcore_axis_name="s")
    f = functools.partial(
        pl.kernel,
        out_type=jax.ShapeDtypeStruct((2, NP, DP), jnp.float32),
        mesh=mesh,
        compiler_params=pltpu.CompilerParams(
            needs_layout_passes=False, use_tc_tiling_on_sc=False),
        scratch_types=[
            pltpu.VMEM_SHARED((NP, DP), jnp.float32),
            pltpu.VMEM((KC,), jnp.int32),
            pltpu.VMEM((KC,), jnp.int32),
            pltpu.VMEM((KC, DP), jnp.float32),
            pltpu.VMEM((N, 4), jnp.float32),
            pltpu.SemaphoreType.DMA,
        ],
    )(_sc_body)
    return f(src1d, dst1d, xlA, xlB, adA, adB, zin)


def _halves(acc_ref, xlA_ref, xlB_ref, adA_ref, adB_ref, e4):
    outs = []
    for half, (xl_ref, ad_ref) in enumerate(((xlA_ref, adA_ref),
                                             (xlB_ref, adB_ref))):
        acc = acc_ref[half]
        xl = xl_ref[...]
        al = xl[:, 64:68] + ad_ref[...]
        ws = jnp.exp(jnp.where(al >= 0.0, al, 0.2 * al))
        den = acc[:, 64:68] + ws + 1e-16
        wsx = jnp.dot(ws, e4, precision=_HI, preferred_element_type=jnp.float32)
        denx = jnp.dot(den, e4, precision=_HI, preferred_element_type=jnp.float32)
        outs.append((acc[:, 0:64] + wsx * xl[:, 0:64]) / denx)
    return outs


def _f1_body(acc_ref, xlA_ref, xlB_ref, adA_ref, adB_ref, e4_ref, b_ref,
             out_ref):
    outA, outB = _halves(acc_ref, xlA_ref, xlB_ref, adA_ref, adB_ref,
                         e4_ref[...])
    h = jnp.concatenate([outA, outB], axis=1) + b_ref[...]
    out_ref[...] = jnp.where(h > 0.0, h, jnp.exp(jnp.minimum(h, 0.0)) - 1.0)


def _f2_body(acc_ref, xlA_ref, xlB_ref, adA_ref, adB_ref, e4_ref, m4_ref,
             b_ref, out_ref):
    outA, outB = _halves(acc_ref, xlA_ref, xlB_ref, adA_ref, adB_ref,
                         e4_ref[...])
    m4 = m4_ref[...]
    hs = (jnp.dot(outA, m4, precision=_HI, preferred_element_type=jnp.float32)
          + jnp.dot(outB, m4, precision=_HI, preferred_element_type=jnp.float32))
    t = hs * 0.125 + b_ref[...]
    mx = jnp.max(t, axis=1, keepdims=True)
    lse = jnp.log(jnp.sum(jnp.exp(t - mx), axis=1, keepdims=True))
    out_ref[...] = t - mx - lse


def _fin_specs(extra_in, out_w):
    in_specs = [
        pl.BlockSpec((2, BN, DP), lambda i: (0, i, 0)),
        pl.BlockSpec((BN, DP), lambda i: (i, 0)),
        pl.BlockSpec((BN, DP), lambda i: (i, 0)),
        pl.BlockSpec((BN, 4), lambda i: (i, 0)),
        pl.BlockSpec((BN, 4), lambda i: (i, 0)),
        pl.BlockSpec((4, 64), lambda i: (0, 0)),
    ] + extra_in
    return dict(
        grid=(GRID,),
        in_specs=in_specs,
        out_specs=pl.BlockSpec((BN, out_w), lambda i: (i, 0)),
        out_shape=jax.ShapeDtypeStruct((N, out_w), jnp.float32),
    )


def kernel(x, edge_index, W1, att_src1, att_dst1, b1, W2, att_src2, att_dst2,
           b2):
    src1d = edge_index[0].astype(jnp.int32)
    dst1d = edge_index[1].astype(jnp.int32)
    zin = jnp.zeros((NP, DP), jnp.float32)

    onehot = (jnp.arange(H)[None, :] == (jnp.arange(D)[:, None] // C)).astype(
        jnp.float32)
    e4 = (jnp.arange(4)[:, None] == (jnp.arange(64)[None, :] // 16)).astype(
        jnp.float32)
    m4 = (jnp.arange(16)[None, :] == (jnp.arange(64)[:, None] % 16)).astype(
        jnp.float32)

    def asd_of(att_s, att_d):
        a_s = att_s.reshape(D)[:, None] * onehot
        a_d = att_d.reshape(D)[:, None] * onehot
        return jnp.concatenate([a_s, a_d], axis=1)

    xlA, xlB, adA, adB = _mm_call(x, W1, asd_of(att_src1, att_dst1))
    acc = _sc_call(src1d, dst1d, xlA, xlB, adA, adB, zin)[:, :N, :]
    h1 = pl.pallas_call(
        _f1_body,
        **_fin_specs([pl.BlockSpec((1, D), lambda i: (0, 0))], D),
    )(acc, xlA, xlB, adA, adB, e4, b1.reshape(1, D))

    xlA2, xlB2, adA2, adB2 = _mm_call(h1, W2, asd_of(att_src2, att_dst2))
    acc2 = _sc_call(src1d, dst1d, xlA2, xlB2, adA2, adB2, zin)[:, :N, :]
    out = pl.pallas_call(
        _f2_body,
        **_fin_specs([pl.BlockSpec((64, 16), lambda i: (0, 0)),
                      pl.BlockSpec((1, 16), lambda i: (0, 0))], 16),
    )(acc2, xlA2, xlB2, adA2, adB2, e4, m4, b2.reshape(1, 16))
    return out


# speedup vs baseline: 65.3093x; 1.0002x over previous
import functools

import jax
import jax.numpy as jnp
from jax import lax
from jax.experimental import pallas as pl
from jax.experimental.pallas import tpu as pltpu
from jax.experimental.pallas import tpu_sc as plsc

N = 10000
NP = 10240
E = 320000
H = 8
C = 16
D = H * C
DP = 68
KC = 80
TILES = 16
ROWS_PT = NP // TILES
E_PT = E // TILES
CHUNKS = E_PT // KC
BN = 2000
GRID = N // BN

_HI = jax.lax.Precision.HIGHEST


def _mm_body(x_ref, w_ref, asd_ref, xlA_ref, xlB_ref, adA_ref, adB_ref):
    x = x_ref[...]
    xl = jnp.dot(x, w_ref[...], precision=_HI, preferred_element_type=jnp.float32)
    sd = jnp.dot(xl, asd_ref[...], precision=_HI, preferred_element_type=jnp.float32)
    xlA_ref[...] = jnp.concatenate([xl[:, 0:64], sd[:, 0:4]], axis=1)
    xlB_ref[...] = jnp.concatenate([xl[:, 64:128], sd[:, 4:8]], axis=1)
    adA_ref[...] = sd[:, 8:12]
    adB_ref[...] = sd[:, 12:16]


def _mm_call(x, W, ASD):
    return pl.pallas_call(
        _mm_body,
        grid=(GRID,),
        in_specs=[
            pl.BlockSpec((BN, D), lambda i: (i, 0)),
            pl.BlockSpec((D, D), lambda i: (0, 0)),
            pl.BlockSpec((D, 16), lambda i: (0, 0)),
        ],
        out_specs=[
            pl.BlockSpec((BN, DP), lambda i: (i, 0)),
            pl.BlockSpec((BN, DP), lambda i: (i, 0)),
            pl.BlockSpec((BN, 4), lambda i: (i, 0)),
            pl.BlockSpec((BN, 4), lambda i: (i, 0)),
        ],
        out_shape=[
            jax.ShapeDtypeStruct((N, DP), jnp.float32),
            jax.ShapeDtypeStruct((N, DP), jnp.float32),
            jax.ShapeDtypeStruct((N, 4), jnp.float32),
            jax.ShapeDtypeStruct((N, 4), jnp.float32),
        ],
    )(x, W, ASD)


def _sc_body(src_hbm, dst_hbm, xlA_hbm, xlB_hbm, adA_hbm, adB_hbm, zin_hbm,
             out_hbm, acc, srcj, dstj, msg, adtab, gsem):
    cid = lax.axis_index("c")
    sid = lax.axis_index("s")
    r0 = sid * ROWS_PT
    pltpu.sync_copy(zin_hbm.at[pl.ds(r0, ROWS_PT)], acc.at[pl.ds(r0, ROWS_PT)])

    iota = lax.iota(jnp.int32, 16)
    i_div4 = iota // 4
    i_mod4 = iota & 3
    wcols = 64 + i_mod4
    four16 = jnp.full((16,), 4, jnp.int32)
    one16 = jnp.full((16,), 1, jnp.int32)

    def run(xl_hbm, ad_hbm):
        pltpu.sync_copy(ad_hbm, adtab)
        plsc.subcore_barrier()
        e_base = sid * E_PT

        def chunk(t, carry):
            eb = e_base + t * KC
            pltpu.sync_copy(src_hbm.at[pl.ds(eb, KC)], srcj)
            pltpu.sync_copy(dst_hbm.at[pl.ds(eb, KC)], dstj)
            pltpu.async_copy(xl_hbm.at[srcj], msg, gsem).wait()

            @functools.partial(plsc.parallel_loop, 0, KC // 4, unroll=8)
            def _wgrp(g):
                i0 = i_div4 + g * four16
                nodes = plsc.load_gather(dstj, [i0])
                a_s = plsc.load_gather(msg, [i0, wcols])
                a_d = plsc.load_gather(adtab, [nodes, i_mod4])
                al = a_s + a_d
                al = jnp.where(al >= 0.0, al, 0.2 * al)
                plsc.store_scatter(msg, [i0, wcols], jnp.exp(al))

            @functools.partial(plsc.parallel_loop, 0, KC, unroll=8)
            def _scale(k):
                k16 = one16 * k
                for h in range(4):
                    wv = plsc.load_gather(
                        msg, [k16, jnp.full((16,), 64 + h, jnp.int32)])
                    ch = iota + 16 * h
                    feat = plsc.load_gather(msg, [k16, ch])
                    plsc.store_scatter(msg, [k16, ch], feat * wv)
            pltpu.sync_copy(msg, acc.at[dstj], add=True)
            return carry

        lax.fori_loop(0, CHUNKS, chunk, 0)

    @pl.when(cid == 0)
    def _():
        run(xlA_hbm, adA_hbm)

    @pl.when(cid == 1)
    def _():
        run(xlB_hbm, adB_hbm)

    plsc.subcore_barrier()
    pltpu.sync_copy(acc.at[pl.ds(r0, ROWS_PT)],
                    out_hbm.at[cid].at[pl.ds(r0, ROWS_PT)])


def _sc_call(src1d, dst1d, xlA, xlB, adA, adB, zin):
    mesh = plsc.VectorSubcoreMesh(core_axis_name="c", subcore_axis_name="s")
    f = functools.partial(
        pl.kernel,
        out_type=jax.ShapeDtypeStruct((2, NP, DP), jnp.float32),
        mesh=mesh,
        compiler_params=pltpu.CompilerParams(
            needs_layout_passes=False, use_tc_tiling_on_sc=False),
        scratch_types=[
            pltpu.VMEM_SHARED((NP, DP), jnp.float32),
            pltpu.VMEM((KC,), jnp.int32),
            pltpu.VMEM((KC,), jnp.int32),
            pltpu.VMEM((KC, DP), jnp.float32),
            pltpu.VMEM((N, 4), jnp.float32),
            pltpu.SemaphoreType.DMA,
        ],
    )(_sc_body)
    return f(src1d, dst1d, xlA, xlB, adA, adB, zin)


def _halves(acc_ref, xlA_ref, xlB_ref, adA_ref, adB_ref, e4):
    outs = []
    for half, (xl_ref, ad_ref) in enumerate(((xlA_ref, adA_ref),
                                             (xlB_ref, adB_ref))):
        acc = acc_ref[half]
        xl = xl_ref[...]
        al = xl[:, 64:68] + ad_ref[...]
        ws = jnp.exp(jnp.where(al >= 0.0, al, 0.2 * al))
        den = acc[:, 64:68] + ws + 1e-16
        wsx = jnp.dot(ws, e4, precision=_HI, preferred_element_type=jnp.float32)
        denx = jnp.dot(den, e4, precision=_HI, preferred_element_type=jnp.float32)
        outs.append((acc[:, 0:64] + wsx * xl[:, 0:64]) / denx)
    return outs


def _f1_body(acc_ref, xlA_ref, xlB_ref, adA_ref, adB_ref, e4_ref, b_ref,
             out_ref):
    outA, outB = _halves(acc_ref, xlA_ref, xlB_ref, adA_ref, adB_ref,
                         e4_ref[...])
    h = jnp.concatenate([outA, outB], axis=1) + b_ref[...]
    out_ref[...] = jnp.where(h > 0.0, h, jnp.exp(jnp.minimum(h, 0.0)) - 1.0)


def _f2_body(acc_ref, xlA_ref, xlB_ref, adA_ref, adB_ref, e4_ref, m4_ref,
             b_ref, out_ref):
    outA, outB = _halves(acc_ref, xlA_ref, xlB_ref, adA_ref, adB_ref,
                         e4_ref[...])
    m4 = m4_ref[...]
    hs = (jnp.dot(outA, m4, precision=_HI, preferred_element_type=jnp.float32)
          + jnp.dot(outB, m4, precision=_HI, preferred_element_type=jnp.float32))
    t = hs * 0.125 + b_ref[...]
    mx = jnp.max(t, axis=1, keepdims=True)
    lse = jnp.log(jnp.sum(jnp.exp(t - mx), axis=1, keepdims=True))
    out_ref[...] = t - mx - lse


def _fin_specs(extra_in, out_w):
    in_specs = [
        pl.BlockSpec((2, BN, DP), lambda i: (0, i, 0)),
        pl.BlockSpec((BN, DP), lambda i: (i, 0)),
        pl.BlockSpec((BN, DP), lambda i: (i, 0)),
        pl.BlockSpec((BN, 4), lambda i: (i, 0)),
        pl.BlockSpec((BN, 4), lambda i: (i, 0)),
        pl.BlockSpec((4, 64), lambda i: (0, 0)),
    ] + extra_in
    return dict(
        grid=(GRID,),
        in_specs=in_specs,
        out_specs=pl.BlockSpec((BN, out_w), lambda i: (i, 0)),
        out_shape=jax.ShapeDtypeStruct((N, out_w), jnp.float32),
    )


def kernel(x, edge_index, W1, att_src1, att_dst1, b1, W2, att_src2, att_dst2,
           b2):
    src1d = edge_index[0].astype(jnp.int32)
    dst1d = edge_index[1].astype(jnp.int32)
    zin = jnp.zeros((NP, DP), jnp.float32)

    onehot = (jnp.arange(H)[None, :] == (jnp.arange(D)[:, None] // C)).astype(
        jnp.float32)
    e4 = (jnp.arange(4)[:, None] == (jnp.arange(64)[None, :] // 16)).astype(
        jnp.float32)
    m4 = (jnp.arange(16)[None, :] == (jnp.arange(64)[:, None] % 16)).astype(
        jnp.float32)

    def asd_of(att_s, att_d):
        a_s = att_s.reshape(D)[:, None] * onehot
        a_d = att_d.reshape(D)[:, None] * onehot
        return jnp.concatenate([a_s, a_d], axis=1)

    xlA, xlB, adA, adB = _mm_call(x, W1, asd_of(att_src1, att_dst1))
    acc = _sc_call(src1d, dst1d, xlA, xlB, adA, adB, zin)[:, :N, :]
    h1 = pl.pallas_call(
        _f1_body,
        **_fin_specs([pl.BlockSpec((1, D), lambda i: (0, 0))], D),
    )(acc, xlA, xlB, adA, adB, e4, b1.reshape(1, D))

    xlA2, xlB2, adA2, adB2 = _mm_call(h1, W2, asd_of(att_src2, att_dst2))
    acc2 = _sc_call(src1d, dst1d, xlA2, xlB2, adA2, adB2, zin)[:, :N, :]
    out = pl.pallas_call(
        _f2_body,
        **_fin_specs([pl.BlockSpec((64, 16), lambda i: (0, 0)),
                      pl.BlockSpec((1, 16), lambda i: (0, 0))], 16),
    )(acc2, xlA2, xlB2, adA2, adB2, e4, m4, b2.reshape(1, 16))
    return out
